# baseline (speedup 1.0000x reference)
import functools

import jax
import jax.numpy as jnp
from jax import lax
from jax.experimental import pallas as pl
from jax.experimental.pallas import tpu as pltpu
from jax.experimental.pallas import tpu_sc as plsc

N, D, E, DFF = 2048, 1024, 8, 1024
NL = N // 2
BM = 128
PR_LO = 2 * NL + E * BM
PR_HI = 2 * N + E * BM
PB_LO = PR_LO // BM
PB_HI = PR_HI // BM

_INTERPRET = False


def _ln(t, g, b, eps=1e-5):
    m = jnp.mean(t, axis=-1, keepdims=True)
    v = jnp.mean((t - m) ** 2, axis=-1, keepdims=True)
    return (t - m) * jax.lax.rsqrt(v + eps) * g + b


def _route_compute(h, Wr, br, nt, pb):
    na = 2 * nt
    logits = jnp.dot(h, Wr, preferred_element_type=jnp.float32) + br
    iota = jax.lax.broadcasted_iota(jnp.int32, (nt, E), 1)
    m0 = jnp.max(logits, axis=-1, keepdims=True)
    i0 = jnp.min(jnp.where(logits == m0, iota, E), axis=-1, keepdims=True)
    masked = jnp.where(iota == i0, -jnp.inf, logits)
    m1 = jnp.max(masked, axis=-1, keepdims=True)
    i1 = jnp.min(jnp.where(masked == m1, iota, E), axis=-1, keepdims=True)
    g0 = 1.0 / (1.0 + jnp.exp(m1 - m0))
    g1 = 1.0 - g0
    ecol = jnp.concatenate([i0, i1], axis=0)
    A = (ecol == jax.lax.broadcasted_iota(jnp.int32, (na, E), 1)).astype(jnp.int32)
    c = A
    k = 1
    while k < na:
        shifted = jnp.concatenate(
            [jnp.zeros((k, E), jnp.int32), c[: na - k, :]], axis=0)
        c = c + shifted
        k *= 2
    excl = c - A
    rank = jnp.sum(excl * A, axis=-1, keepdims=True)
    counts = c[na - 1 : na, :]
    p = ((counts + (BM - 1)) // BM) * BM
    tri = (jax.lax.broadcasted_iota(jnp.int32, (E, E), 0)
           < jax.lax.broadcasted_iota(jnp.int32, (E, E), 1)).astype(jnp.float32)
    off = jnp.dot(p.astype(jnp.float32), tri,
                  preferred_element_type=jnp.float32).astype(jnp.int32)
    dest = jnp.sum(A * off, axis=-1, keepdims=True) + rank
    cum = off + p
    tot = jnp.sum(p)
    bstart = jax.lax.broadcasted_iota(jnp.int32, (pb, 1), 0) * BM
    ge = jnp.sum((cum <= bstart).astype(jnp.int32), axis=-1, keepdims=True)
    be = jnp.where(bstart < tot, ge, -1)
    return dest, be, g0, g1


def _down_body(x_ref, pw_ref, W_ref, b_ref, g_ref, be_ref, lng_ref, lnb_ref,
               Wr_ref, br_ref,
               xp2_ref, xd_ref, h_ref, dest_ref, bemap_ref, g0_ref, g1_ref):
    x = x_ref[...]
    xg = x.reshape(NL, 2, D)
    xeven = xg[:, 0, :]
    xodd = xg[:, 1, :]
    xp2_ref[0] = xeven
    xp2_ref[1] = xodd
    w = jax.nn.softmax(pw_ref[...], axis=-1)
    xp = xeven * w[:, 0:1] + xodd * w[:, 1:2]
    t = jnp.dot(xp, W_ref[...], preferred_element_type=jnp.float32) + b_ref[...]
    xd = jnp.maximum(_ln(t, g_ref[...], be_ref[...]), 0.0)
    xd_ref[...] = xd
    h = _ln(xd, lng_ref[...], lnb_ref[...])
    h_ref[...] = h
    dest, be, g0, g1 = _route_compute(h, Wr_ref[...], br_ref[...], NL, PB_LO)
    dest_ref[...] = dest
    bemap_ref[...] = be
    g0_ref[...] = g0
    g1_ref[...] = g1


def _route_body(nt, pb, h_ref, Wr_ref, br_ref, dest_ref, be_ref, g0_ref, g1_ref):
    dest, be, g0, g1 = _route_compute(h_ref[...], Wr_ref[...], br_ref[...],
                                      nt, pb)
    dest_ref[...] = dest
    be_ref[...] = be
    g0_ref[...] = g0
    g1_ref[...] = g1


def _ffn_body(be_ref, x_ref, W1_ref, b1_ref, W2_ref, b2_ref, out_ref):
    b = pl.program_id(0)

    @pl.when(be_ref[b] >= 0)
    def _():
        hidden = jnp.maximum(
            jnp.dot(x_ref[...], W1_ref[0], preferred_element_type=jnp.float32)
            + b1_ref[0], 0.0)
        out_ref[...] = (
            jnp.dot(hidden, W2_ref[0], preferred_element_type=jnp.float32)
            + b2_ref[0])


def _up_body(xd_ref, y0_ref, y1_ref, g0_ref, g1_ref, W1_ref, b1_ref, g_ref,
             be_ref, W2_ref, b2_ref, pos_ref, xp2_ref, lng_ref, lnb_ref,
             s_ref, xc2_ref, h2_ref):
    xl = xd_ref[...] + g0_ref[...] * y0_ref[...] + g1_ref[...] * y1_ref[...]
    t1 = jnp.dot(xl, W1_ref[...], preferred_element_type=jnp.float32) + b1_ref[...]
    t1 = jnp.maximum(_ln(t1, g_ref[...], be_ref[...]), 0.0)
    t2 = jnp.dot(t1, W2_ref[...], preferred_element_type=jnp.float32) + b2_ref[...]
    s = s_ref[0, 0]
    xce = xp2_ref[0] + s * (t2[:, :D] + pos_ref[0:1, :])
    xco = xp2_ref[1] + s * (t2[:, D:] + pos_ref[1:2, :])
    xc2_ref[0] = xce
    xc2_ref[1] = xco
    h2_ref[0] = _ln(xce, lng_ref[...], lnb_ref[...])
    h2_ref[1] = _ln(xco, lng_ref[...], lnb_ref[...])


def _final_body(xp2_ref, xc2_ref, g0e_ref, g0o_ref, g1e_ref, g1o_ref,
                y0e_ref, y0o_ref, y1e_ref, y1o_ref, s_ref, out_ref):
    s = s_ref[0, 0]
    oute = xp2_ref[0] + s * (xc2_ref[0]
                             + g0e_ref[...] * y0e_ref[...]
                             + g1e_ref[...] * y1e_ref[...])
    outo = xp2_ref[1] + s * (xc2_ref[1]
                             + g0o_ref[...] * y0o_ref[...]
                             + g1o_ref[...] * y1o_ref[...])
    out_ref[:, 0, :] = oute
    out_ref[:, 1, :] = outo


def _sc_dispatch(h, dest, pr):
    nt = h.shape[0]
    info = plsc.get_sparse_core_info()
    nc = info.num_cores
    nw = nc * info.num_subcores
    ch = nt // nw
    sub = min(ch, 64)
    mesh = plsc.VectorSubcoreMesh(core_axis_name="c", subcore_axis_name="s")

    @functools.partial(
        pl.kernel, mesh=mesh,
        out_type=jax.ShapeDtypeStruct((pr, D), h.dtype),
        scratch_types=[
            pltpu.VMEM((sub,), jnp.int32),
            pltpu.VMEM((sub,), jnp.int32),
            pltpu.VMEM((sub, D), h.dtype),
            pltpu.SemaphoreType.DMA,
            pltpu.SemaphoreType.DMA,
        ],
    )
    def k(h_hbm, dest_hbm, buf_hbm, i0_v, i1_v, rows_v, s0, s1):
        wid = lax.axis_index("s") * nc + lax.axis_index("c")
        base = wid * ch
        for j in range(ch // sub):
            tbase = base + j * sub
            pltpu.sync_copy(dest_hbm.at[pl.ds(tbase, sub)], i0_v)
            pltpu.sync_copy(dest_hbm.at[pl.ds(nt + tbase, sub)], i1_v)
            pltpu.sync_copy(h_hbm.at[pl.ds(tbase, sub)], rows_v)
            c0 = pltpu.async_copy(rows_v, buf_hbm.at[i0_v], s0)
            c1 = pltpu.async_copy(rows_v, buf_hbm.at[i1_v], s1)
            c0.wait()
            c1.wait()

    return k(h, dest)


def _sc_combine(y, dest):
    na = dest.shape[0]
    info = plsc.get_sparse_core_info()
    nc = info.num_cores
    nw = nc * info.num_subcores
    ch = na // nw
    sub = min(ch, 64)
    mesh = plsc.VectorSubcoreMesh(core_axis_name="c", subcore_axis_name="s")

    @functools.partial(
        pl.kernel, mesh=mesh,
        out_type=jax.ShapeDtypeStruct((na, D), y.dtype),
        scratch_types=[
            pltpu.VMEM((sub,), jnp.int32),
            pltpu.VMEM((sub, D), y.dtype),
            pltpu.SemaphoreType.DMA,
        ],
    )
    def k(y_hbm, dest_hbm, out_hbm, idx_v, rows_v, sem):
        wid = lax.axis_index("s") * nc + lax.axis_index("c")
        base = wid * ch
        for j in range(ch // sub):
            abase = base + j * sub
            pltpu.sync_copy(dest_hbm.at[pl.ds(abase, sub)], idx_v)
            pltpu.async_copy(y_hbm.at[idx_v], rows_v, sem).wait()
            pltpu.sync_copy(rows_v, out_hbm.at[pl.ds(abase, sub)])

    return k(y, dest)


def _route_call(h, Wr, br, nt, pb):
    f32, i32 = jnp.float32, jnp.int32
    return pl.pallas_call(
        functools.partial(_route_body, nt, pb),
        out_shape=[jax.ShapeDtypeStruct((2 * nt, 1), i32),
                   jax.ShapeDtypeStruct((pb, 1), i32),
                   jax.ShapeDtypeStruct((nt, 1), f32),
                   jax.ShapeDtypeStruct((nt, 1), f32)],
        interpret=_INTERPRET,
    )(h, Wr, br)


def _ffn_call(be, buf, W1, b1, W2, b2, pr, pb):
    return pl.pallas_call(
        _ffn_body,
        grid_spec=pltpu.PrefetchScalarGridSpec(
            num_scalar_prefetch=1,
            grid=(pb,),
            in_specs=[
                pl.BlockSpec((BM, D), lambda b, be: (b, 0)),
                pl.BlockSpec((1, D, DFF),
                             lambda b, be: (jnp.maximum(be[b], 0), 0, 0)),
                pl.BlockSpec((1, 1, DFF),
                             lambda b, be: (jnp.maximum(be[b], 0), 0, 0)),
                pl.BlockSpec((1, DFF, D),
                             lambda b, be: (jnp.maximum(be[b], 0), 0, 0)),
                pl.BlockSpec((1, 1, D),
                             lambda b, be: (jnp.maximum(be[b], 0), 0, 0)),
            ],
            out_specs=pl.BlockSpec((BM, D), lambda b, be: (b, 0)),
        ),
        out_shape=jax.ShapeDtypeStruct((pr, D), jnp.float32),
        interpret=_INTERPRET,
    )(be, buf, W1, b1.reshape(E, 1, DFF), W2, b2.reshape(E, 1, D))


def kernel(x, down_pool_w, down_W, down_b, down_g, down_beta, lo_ln_g, lo_ln_b,
           lo_Wr, lo_br, lo_W1, lo_b1, lo_W2, lo_b2, up_W1, up_b1, up_g, up_beta,
           up_W2, up_b2, up_pos, hi_ln_g, hi_ln_b, hi_Wr, hi_br, hi_W1, hi_b1,
           hi_W2, hi_b2, scale_lo, scale_hi):
    f32, i32 = jnp.float32, jnp.int32
    x2 = x.reshape(N, D)
    r2 = lambda v: v.reshape(1, -1)
    sL = jnp.reshape(scale_lo, (1, 1)).astype(f32)
    sH = jnp.reshape(scale_hi, (1, 1)).astype(f32)

    xp2, xd, h_lo, dest_lo, be_lo, g0_lo, g1_lo = pl.pallas_call(
        _down_body,
        out_shape=[jax.ShapeDtypeStruct((2, NL, D), f32),
                   jax.ShapeDtypeStruct((NL, D), f32),
                   jax.ShapeDtypeStruct((NL, D), f32),
                   jax.ShapeDtypeStruct((2 * NL, 1), i32),
                   jax.ShapeDtypeStruct((PB_LO, 1), i32),
                   jax.ShapeDtypeStruct((NL, 1), f32),
                   jax.ShapeDtypeStruct((NL, 1), f32)],
        interpret=_INTERPRET,
    )(x2, down_pool_w, down_W, r2(down_b), r2(down_g), r2(down_beta),
      r2(lo_ln_g), r2(lo_ln_b), lo_Wr, r2(lo_br))

    buf_lo = _sc_dispatch(h_lo, dest_lo.reshape(-1), PR_LO)
    y_lo = _ffn_call(be_lo.reshape(-1), buf_lo, lo_W1, lo_b1, lo_W2, lo_b2,
                     PR_LO, PB_LO)
    y2_lo = _sc_combine(y_lo, dest_lo.reshape(-1))

    BU = 256
    TU = NL // BU
    blk = lambda: pl.BlockSpec((BU, D), lambda t: (t, 0))
    blk1 = lambda: pl.BlockSpec((BU, 1), lambda t: (t, 0))
    pln = lambda: pl.BlockSpec((2, BU, D), lambda t: (0, t, 0))
    xc2, h2 = pl.pallas_call(
        _up_body,
        grid=(TU,),
        in_specs=[
            blk(),
            pl.BlockSpec((BU, D), lambda t: (t, 0)),
            pl.BlockSpec((BU, D), lambda t: (NL // BU + t, 0)),
            blk1(), blk1(),
            _full((D, 2 * D)), _full((1, 2 * D)), _full((1, 2 * D)),
            _full((1, 2 * D)), _full((2 * D, 2 * D)), _full((1, 2 * D)),
            _full((2, D)),
            pln(),
            _full((1, D)), _full((1, D)), _full((1, 1)),
        ],
        out_specs=[pln(), pln()],
        out_shape=[jax.ShapeDtypeStruct((2, NL, D), f32)] * 2,
        interpret=_INTERPRET,
    )(xd, y2_lo, y2_lo, g0_lo, g1_lo, up_W1, r2(up_b1), r2(up_g), r2(up_beta),
      up_W2, r2(up_b2), up_pos, xp2, r2(hi_ln_g), r2(hi_ln_b), sL)

    hp = h2.reshape(N, D)
    dest_hi, be_hi, g0_hi, g1_hi = _route_call(hp, hi_Wr, r2(hi_br), N, PB_HI)
    buf_hi = _sc_dispatch(hp, dest_hi.reshape(-1), PR_HI)
    y_hi = _ffn_call(be_hi.reshape(-1), buf_hi, hi_W1, hi_b1, hi_W2, hi_b2,
                     PR_HI, PB_HI)
    y2_hi = _sc_combine(y_hi, dest_hi.reshape(-1))

    BF = 256
    TF = NL // BF
    fb = lambda off: pl.BlockSpec((BF, D), lambda t: (off + t, 0))
    fb1 = lambda off: pl.BlockSpec((BF, 1), lambda t: (off + t, 0))
    fpl = lambda: pl.BlockSpec((2, BF, D), lambda t: (0, t, 0))
    outp = pl.pallas_call(
        _final_body,
        grid=(TF,),
        in_specs=[
            fpl(), fpl(),
            fb1(0), fb1(NL // BF), fb1(0), fb1(NL // BF),
            fb(0), fb(NL // BF), fb(2 * (NL // BF)), fb(3 * (NL // BF)),
            _full((1, 1)),
        ],
        out_specs=pl.BlockSpec((BF, 2, D), lambda t: (t, 0, 0)),
        out_shape=jax.ShapeDtypeStruct((NL, 2, D), f32),
        interpret=_INTERPRET,
    )(xp2, xc2, g0_hi, g0_hi, g1_hi, g1_hi, y2_hi, y2_hi, y2_hi, y2_hi, sH)

    return outp.reshape(1, N, D)


def _full(shape):
    nd = len(shape)
    return pl.BlockSpec(shape, lambda *_: (0,) * nd)


# speedup vs baseline: 1.0850x; 1.0850x over previous
import functools

import jax
import jax.numpy as jnp
from jax import lax
from jax.experimental import pallas as pl
from jax.experimental.pallas import tpu as pltpu
from jax.experimental.pallas import tpu_sc as plsc

N, D, E, DFF = 2048, 1024, 8, 1024
NL = N // 2
BM = 128
PR_LO = 2 * NL + E * BM
PR_HI = 2 * N + E * BM
PB_LO = PR_LO // BM
PB_HI = PR_HI // BM

_INTERPRET = False


def _ln(t, g, b, eps=1e-5):
    m = jnp.mean(t, axis=-1, keepdims=True)
    v = jnp.mean((t - m) ** 2, axis=-1, keepdims=True)
    return (t - m) * jax.lax.rsqrt(v + eps) * g + b


D2 = D // 2


def _pack_bf16(x):
    d2 = x.shape[-1] // 2
    u = jax.lax.bitcast_convert_type(x, jnp.uint32)
    r = u + jnp.uint32(0x7FFF) + ((u >> 16) & jnp.uint32(1))
    lo = r[:, :d2] >> 16
    hi = r[:, d2:] & jnp.uint32(0xFFFF0000)
    return jax.lax.bitcast_convert_type(lo | hi, jnp.int32)


def _unpack_bf16(p):
    u = jax.lax.bitcast_convert_type(p, jnp.uint32)
    f_lo = jax.lax.bitcast_convert_type(u << 16, jnp.float32)
    f_hi = jax.lax.bitcast_convert_type(u & jnp.uint32(0xFFFF0000), jnp.float32)
    return jnp.concatenate([f_lo, f_hi], axis=-1)


def _route_compute(logits, nt, pb):
    na = 2 * nt
    iota = jax.lax.broadcasted_iota(jnp.int32, (nt, E), 1)
    m0 = jnp.max(logits, axis=-1, keepdims=True)
    i0 = jnp.min(jnp.where(logits == m0, iota, E), axis=-1, keepdims=True)
    masked = jnp.where(iota == i0, -jnp.inf, logits)
    m1 = jnp.max(masked, axis=-1, keepdims=True)
    i1 = jnp.min(jnp.where(masked == m1, iota, E), axis=-1, keepdims=True)
    g0 = 1.0 / (1.0 + jnp.exp(m1 - m0))
    g1 = 1.0 - g0
    ecol = jnp.concatenate([i0, i1], axis=0)
    A = (ecol == jax.lax.broadcasted_iota(jnp.int32, (na, E), 1)).astype(jnp.int32)
    c = A
    k = 1
    while k < na:
        shifted = jnp.concatenate(
            [jnp.zeros((k, E), jnp.int32), c[: na - k, :]], axis=0)
        c = c + shifted
        k *= 2
    excl = c - A
    rank = jnp.sum(excl * A, axis=-1, keepdims=True)
    counts = c[na - 1 : na, :]
    p = ((counts + (BM - 1)) // BM) * BM
    tri = (jax.lax.broadcasted_iota(jnp.int32, (E, E), 0)
           < jax.lax.broadcasted_iota(jnp.int32, (E, E), 1)).astype(jnp.float32)
    off = jnp.dot(p.astype(jnp.float32), tri,
                  preferred_element_type=jnp.float32).astype(jnp.int32)
    dest = jnp.sum(A * off, axis=-1, keepdims=True) + rank
    cum = off + p
    tot = jnp.sum(p)
    bstart = jax.lax.broadcasted_iota(jnp.int32, (pb, 1), 0) * BM
    ge = jnp.sum((cum <= bstart).astype(jnp.int32), axis=-1, keepdims=True)
    be = jnp.where(bstart < tot, ge, -1)
    return dest, be, g0, g1


def _down_body(x_ref, pw_ref, W_ref, b_ref, g_ref, be_ref, lng_ref, lnb_ref,
               Wr_ref, br_ref,
               xp2_ref, xd_ref, h_ref, dest_ref, bemap_ref, g0_ref, g1_ref):
    x = x_ref[...]
    xg = x.reshape(NL, 2, D)
    xeven = xg[:, 0, :]
    xodd = xg[:, 1, :]
    xp2_ref[0] = xeven
    xp2_ref[1] = xodd
    w = jax.nn.softmax(pw_ref[...], axis=-1)
    xp = xeven * w[:, 0:1] + xodd * w[:, 1:2]
    t = jnp.dot(xp, W_ref[...], preferred_element_type=jnp.float32) + b_ref[...]
    xd = jnp.maximum(_ln(t, g_ref[...], be_ref[...]), 0.0)
    xd_ref[...] = xd
    h = _ln(xd, lng_ref[...], lnb_ref[...])
    h_ref[...] = _pack_bf16(h)
    logits = jnp.dot(h, Wr_ref[...], preferred_element_type=jnp.float32) + br_ref[...]
    dest, be, g0, g1 = _route_compute(logits, NL, PB_LO)
    dest_ref[...] = dest
    bemap_ref[...] = be
    g0_ref[...] = g0
    g1_ref[...] = g1


def _route_body(nt, pb, lg_ref, dest_ref, be_ref, g0_ref, g1_ref):
    dest, be, g0, g1 = _route_compute(lg_ref[...], nt, pb)
    dest_ref[...] = dest
    be_ref[...] = be
    g0_ref[...] = g0
    g1_ref[...] = g1


def _ffn_body(be_ref, x_ref, W1_ref, b1_ref, W2_ref, b2_ref, out_ref):
    b = pl.program_id(0)

    @pl.when(be_ref[b] >= 0)
    def _():
        xin = _unpack_bf16(x_ref[...])
        hidden = jnp.maximum(
            jnp.dot(xin, W1_ref[0], preferred_element_type=jnp.float32)
            + b1_ref[0], 0.0)
        out_ref[...] = _pack_bf16(
            jnp.dot(hidden, W2_ref[0], preferred_element_type=jnp.float32)
            + b2_ref[0])


def _up_body(xd_ref, y0_ref, y1_ref, g0_ref, g1_ref, W1_ref, b1_ref, g_ref,
             be_ref, W2_ref, b2_ref, pos_ref, xp2_ref, lng_ref, lnb_ref,
             Wr_ref, br_ref, s_ref, xc2_ref, h2_ref, lg_ref):
    y0 = _unpack_bf16(y0_ref[...])
    y1 = _unpack_bf16(y1_ref[...])
    xl = xd_ref[...] + g0_ref[...] * y0 + g1_ref[...] * y1
    t1 = jnp.dot(xl, W1_ref[...], preferred_element_type=jnp.float32) + b1_ref[...]
    t1 = jnp.maximum(_ln(t1, g_ref[...], be_ref[...]), 0.0)
    t2 = jnp.dot(t1, W2_ref[...], preferred_element_type=jnp.float32) + b2_ref[...]
    s = s_ref[0, 0]
    xce = xp2_ref[0] + s * (t2[:, :D] + pos_ref[0:1, :])
    xco = xp2_ref[1] + s * (t2[:, D:] + pos_ref[1:2, :])
    xc2_ref[0] = xce
    xc2_ref[1] = xco
    he = _ln(xce, lng_ref[...], lnb_ref[...])
    ho = _ln(xco, lng_ref[...], lnb_ref[...])
    h2_ref[0] = _pack_bf16(he)
    h2_ref[1] = _pack_bf16(ho)
    Wr = Wr_ref[...]
    br = br_ref[...]
    lg_ref[0] = jnp.dot(he, Wr, preferred_element_type=jnp.float32) + br
    lg_ref[1] = jnp.dot(ho, Wr, preferred_element_type=jnp.float32) + br


def _final_body(xp2_ref, xc2_ref, g0e_ref, g0o_ref, g1e_ref, g1o_ref,
                y0e_ref, y0o_ref, y1e_ref, y1o_ref, s_ref, out_ref):
    s = s_ref[0, 0]
    oute = xp2_ref[0] + s * (xc2_ref[0]
                             + g0e_ref[...] * _unpack_bf16(y0e_ref[...])
                             + g1e_ref[...] * _unpack_bf16(y1e_ref[...]))
    outo = xp2_ref[1] + s * (xc2_ref[1]
                             + g0o_ref[...] * _unpack_bf16(y0o_ref[...])
                             + g1o_ref[...] * _unpack_bf16(y1o_ref[...]))
    out_ref[:, 0, :] = oute
    out_ref[:, 1, :] = outo


def _sc_dispatch(h, dest, pr):
    nt, wd = h.shape
    info = plsc.get_sparse_core_info()
    nc = info.num_cores
    nw = nc * info.num_subcores
    ch = nt // nw
    sub = min(ch, 64)
    mesh = plsc.VectorSubcoreMesh(core_axis_name="c", subcore_axis_name="s")

    @functools.partial(
        pl.kernel, mesh=mesh,
        out_type=jax.ShapeDtypeStruct((pr, wd), h.dtype),
        scratch_types=[
            pltpu.VMEM((sub,), jnp.int32),
            pltpu.VMEM((sub,), jnp.int32),
            pltpu.VMEM((sub, wd), h.dtype),
            pltpu.SemaphoreType.DMA,
            pltpu.SemaphoreType.DMA,
        ],
    )
    def k(h_hbm, dest_hbm, buf_hbm, i0_v, i1_v, rows_v, s0, s1):
        wid = lax.axis_index("s") * nc + lax.axis_index("c")
        base = wid * ch
        for j in range(ch // sub):
            tbase = base + j * sub
            pltpu.sync_copy(dest_hbm.at[pl.ds(tbase, sub)], i0_v)
            pltpu.sync_copy(dest_hbm.at[pl.ds(nt + tbase, sub)], i1_v)
            pltpu.sync_copy(h_hbm.at[pl.ds(tbase, sub)], rows_v)
            c0 = pltpu.async_copy(rows_v, buf_hbm.at[i0_v], s0)
            c1 = pltpu.async_copy(rows_v, buf_hbm.at[i1_v], s1)
            c0.wait()
            c1.wait()

    return k(h, dest)


def _sc_combine(y, dest):
    na = dest.shape[0]
    wd = y.shape[1]
    info = plsc.get_sparse_core_info()
    nc = info.num_cores
    nw = nc * info.num_subcores
    ch = na // nw
    sub = min(ch, 64)
    mesh = plsc.VectorSubcoreMesh(core_axis_name="c", subcore_axis_name="s")

    @functools.partial(
        pl.kernel, mesh=mesh,
        out_type=jax.ShapeDtypeStruct((na, wd), y.dtype),
        scratch_types=[
            pltpu.VMEM((sub,), jnp.int32),
            pltpu.VMEM((sub, wd), y.dtype),
            pltpu.SemaphoreType.DMA,
        ],
    )
    def k(y_hbm, dest_hbm, out_hbm, idx_v, rows_v, sem):
        wid = lax.axis_index("s") * nc + lax.axis_index("c")
        base = wid * ch
        for j in range(ch // sub):
            abase = base + j * sub
            pltpu.sync_copy(dest_hbm.at[pl.ds(abase, sub)], idx_v)
            pltpu.async_copy(y_hbm.at[idx_v], rows_v, sem).wait()
            pltpu.sync_copy(rows_v, out_hbm.at[pl.ds(abase, sub)])

    return k(y, dest)


def _route_call(lg, nt, pb):
    f32, i32 = jnp.float32, jnp.int32
    return pl.pallas_call(
        functools.partial(_route_body, nt, pb),
        out_shape=[jax.ShapeDtypeStruct((2 * nt, 1), i32),
                   jax.ShapeDtypeStruct((pb, 1), i32),
                   jax.ShapeDtypeStruct((nt, 1), f32),
                   jax.ShapeDtypeStruct((nt, 1), f32)],
        interpret=_INTERPRET,
    )(lg)


def _ffn_call(be, buf, W1, b1, W2, b2, pr, pb):
    return pl.pallas_call(
        _ffn_body,
        grid_spec=pltpu.PrefetchScalarGridSpec(
            num_scalar_prefetch=1,
            grid=(pb,),
            in_specs=[
                pl.BlockSpec((BM, D2), lambda b, be: (b, 0)),
                pl.BlockSpec((1, D, DFF),
                             lambda b, be: (jnp.maximum(be[b], 0), 0, 0)),
                pl.BlockSpec((1, 1, DFF),
                             lambda b, be: (jnp.maximum(be[b], 0), 0, 0)),
                pl.BlockSpec((1, DFF, D),
                             lambda b, be: (jnp.maximum(be[b], 0), 0, 0)),
                pl.BlockSpec((1, 1, D),
                             lambda b, be: (jnp.maximum(be[b], 0), 0, 0)),
            ],
            out_specs=pl.BlockSpec((BM, D2), lambda b, be: (b, 0)),
        ),
        out_shape=jax.ShapeDtypeStruct((pr, D2), jnp.int32),
        interpret=_INTERPRET,
    )(be, buf, W1, b1.reshape(E, 1, DFF), W2, b2.reshape(E, 1, D))


def kernel(x, down_pool_w, down_W, down_b, down_g, down_beta, lo_ln_g, lo_ln_b,
           lo_Wr, lo_br, lo_W1, lo_b1, lo_W2, lo_b2, up_W1, up_b1, up_g, up_beta,
           up_W2, up_b2, up_pos, hi_ln_g, hi_ln_b, hi_Wr, hi_br, hi_W1, hi_b1,
           hi_W2, hi_b2, scale_lo, scale_hi):
    f32, i32 = jnp.float32, jnp.int32
    x2 = x.reshape(N, D)
    r2 = lambda v: v.reshape(1, -1)
    sL = jnp.reshape(scale_lo, (1, 1)).astype(f32)
    sH = jnp.reshape(scale_hi, (1, 1)).astype(f32)

    xp2, xd, h_lo, dest_lo, be_lo, g0_lo, g1_lo = pl.pallas_call(
        _down_body,
        out_shape=[jax.ShapeDtypeStruct((2, NL, D), f32),
                   jax.ShapeDtypeStruct((NL, D), f32),
                   jax.ShapeDtypeStruct((NL, D2), i32),
                   jax.ShapeDtypeStruct((2 * NL, 1), i32),
                   jax.ShapeDtypeStruct((PB_LO, 1), i32),
                   jax.ShapeDtypeStruct((NL, 1), f32),
                   jax.ShapeDtypeStruct((NL, 1), f32)],
        interpret=_INTERPRET,
    )(x2, down_pool_w, down_W, r2(down_b), r2(down_g), r2(down_beta),
      r2(lo_ln_g), r2(lo_ln_b), lo_Wr, r2(lo_br))

    buf_lo = _sc_dispatch(h_lo, dest_lo.reshape(-1), PR_LO)
    y_lo = _ffn_call(be_lo.reshape(-1), buf_lo, lo_W1, lo_b1, lo_W2, lo_b2,
                     PR_LO, PB_LO)
    y2_lo = _sc_combine(y_lo, dest_lo.reshape(-1))

    BU = 256
    TU = NL // BU
    blk = lambda: pl.BlockSpec((BU, D), lambda t: (t, 0))
    blk1 = lambda: pl.BlockSpec((BU, 1), lambda t: (t, 0))
    pln = lambda: pl.BlockSpec((2, BU, D), lambda t: (0, t, 0))
    xc2, h2, lg_hi = pl.pallas_call(
        _up_body,
        grid=(TU,),
        in_specs=[
            blk(),
            pl.BlockSpec((BU, D2), lambda t: (t, 0)),
            pl.BlockSpec((BU, D2), lambda t: (NL // BU + t, 0)),
            blk1(), blk1(),
            _full((D, 2 * D)), _full((1, 2 * D)), _full((1, 2 * D)),
            _full((1, 2 * D)), _full((2 * D, 2 * D)), _full((1, 2 * D)),
            _full((2, D)),
            pln(),
            _full((1, D)), _full((1, D)),
            _full((D, E)), _full((1, E)), _full((1, 1)),
        ],
        out_specs=[pln(),
                   pl.BlockSpec((2, BU, D2), lambda t: (0, t, 0)),
                   pl.BlockSpec((2, BU, E), lambda t: (0, t, 0))],
        out_shape=[jax.ShapeDtypeStruct((2, NL, D), f32),
                   jax.ShapeDtypeStruct((2, NL, D2), i32),
                   jax.ShapeDtypeStruct((2, NL, E), f32)],
        interpret=_INTERPRET,
    )(xd, y2_lo, y2_lo, g0_lo, g1_lo, up_W1, r2(up_b1), r2(up_g), r2(up_beta),
      up_W2, r2(up_b2), up_pos, xp2, r2(hi_ln_g), r2(hi_ln_b),
      hi_Wr, r2(hi_br), sL)

    hp = h2.reshape(N, D2)
    dest_hi, be_hi, g0_hi, g1_hi = _route_call(lg_hi.reshape(N, E), N, PB_HI)
    buf_hi = _sc_dispatch(hp, dest_hi.reshape(-1), PR_HI)
    y_hi = _ffn_call(be_hi.reshape(-1), buf_hi, hi_W1, hi_b1, hi_W2, hi_b2,
                     PR_HI, PB_HI)
    y2_hi = _sc_combine(y_hi, dest_hi.reshape(-1))

    BF = 256
    TF = NL // BF
    fy = lambda off: pl.BlockSpec((BF, D2), lambda t: (off + t, 0))
    fb1 = lambda off: pl.BlockSpec((BF, 1), lambda t: (off + t, 0))
    fpl = lambda: pl.BlockSpec((2, BF, D), lambda t: (0, t, 0))
    outp = pl.pallas_call(
        _final_body,
        grid=(TF,),
        in_specs=[
            fpl(), fpl(),
            fb1(0), fb1(NL // BF), fb1(0), fb1(NL // BF),
            fy(0), fy(NL // BF), fy(2 * (NL // BF)), fy(3 * (NL // BF)),
            _full((1, 1)),
        ],
        out_specs=pl.BlockSpec((BF, 2, D), lambda t: (t, 0, 0)),
        out_shape=jax.ShapeDtypeStruct((NL, 2, D), f32),
        interpret=_INTERPRET,
    )(xp2, xc2, g0_hi, g0_hi, g1_hi, g1_hi, y2_hi, y2_hi, y2_hi, y2_hi, sH)

    return outp.reshape(1, N, D)


def _full(shape):
    nd = len(shape)
    return pl.BlockSpec(shape, lambda *_: (0,) * nd)


# speedup vs baseline: 1.1092x; 1.0223x over previous
import functools

import jax
import jax.numpy as jnp
from jax import lax
from jax.experimental import pallas as pl
from jax.experimental.pallas import tpu as pltpu
from jax.experimental.pallas import tpu_sc as plsc

N, D, E, DFF = 2048, 1024, 8, 1024
NL = N // 2
BM = 128
PR_LO = 2 * NL + E * BM
PR_HI = 2 * N + E * BM
PB_LO = PR_LO // BM
PB_HI = PR_HI // BM

_INTERPRET = False


def _ln(t, g, b, eps=1e-5):
    m = jnp.mean(t, axis=-1, keepdims=True)
    v = jnp.mean((t - m) ** 2, axis=-1, keepdims=True)
    return (t - m) * jax.lax.rsqrt(v + eps) * g + b


D2 = D // 2


def _pack_bf16(x):
    d2 = x.shape[-1] // 2
    u = jax.lax.bitcast_convert_type(x, jnp.uint32)
    r = u + jnp.uint32(0x7FFF) + ((u >> 16) & jnp.uint32(1))
    lo = r[:, :d2] >> 16
    hi = r[:, d2:] & jnp.uint32(0xFFFF0000)
    return jax.lax.bitcast_convert_type(lo | hi, jnp.int32)


def _unpack_bf16(p):
    u = jax.lax.bitcast_convert_type(p, jnp.uint32)
    f_lo = jax.lax.bitcast_convert_type(u << 16, jnp.float32)
    f_hi = jax.lax.bitcast_convert_type(u & jnp.uint32(0xFFFF0000), jnp.float32)
    return jnp.concatenate([f_lo, f_hi], axis=-1)


def _route_compute(logits, nt, pb):
    na = 2 * nt
    iota = jax.lax.broadcasted_iota(jnp.int32, (nt, E), 1)
    m0 = jnp.max(logits, axis=-1, keepdims=True)
    i0 = jnp.min(jnp.where(logits == m0, iota, E), axis=-1, keepdims=True)
    masked = jnp.where(iota == i0, -jnp.inf, logits)
    m1 = jnp.max(masked, axis=-1, keepdims=True)
    i1 = jnp.min(jnp.where(masked == m1, iota, E), axis=-1, keepdims=True)
    g0 = 1.0 / (1.0 + jnp.exp(m1 - m0))
    g1 = 1.0 - g0
    ecol = jnp.concatenate([i0, i1], axis=0)
    A = (ecol == jax.lax.broadcasted_iota(jnp.int32, (na, E), 1)).astype(jnp.int32)
    c = A
    k = 1
    while k < na:
        shifted = jnp.concatenate(
            [jnp.zeros((k, E), jnp.int32), c[: na - k, :]], axis=0)
        c = c + shifted
        k *= 2
    excl = c - A
    rank = jnp.sum(excl * A, axis=-1, keepdims=True)
    counts = c[na - 1 : na, :]
    p = ((counts + (BM - 1)) // BM) * BM
    tri = (jax.lax.broadcasted_iota(jnp.int32, (E, E), 0)
           < jax.lax.broadcasted_iota(jnp.int32, (E, E), 1)).astype(jnp.float32)
    off = jnp.dot(p.astype(jnp.float32), tri,
                  preferred_element_type=jnp.float32).astype(jnp.int32)
    dest = jnp.sum(A * off, axis=-1, keepdims=True) + rank
    cum = off + p
    tot = jnp.sum(p)
    bstart = jax.lax.broadcasted_iota(jnp.int32, (pb, 1), 0) * BM
    ge = jnp.sum((cum <= bstart).astype(jnp.int32), axis=-1, keepdims=True)
    be = jnp.where(bstart < tot, ge, -1)
    return dest, be, g0, g1


def _down_body(x_ref, pw_ref, W_ref, b_ref, g_ref, be_ref, lng_ref, lnb_ref,
               Wr_ref, br_ref,
               xp2_ref, xd_ref, h_ref, dest_ref, bemap_ref, g0_ref, g1_ref):
    x = x_ref[...]
    xg = x.reshape(NL, 2, D)
    xeven = xg[:, 0, :]
    xodd = xg[:, 1, :]
    xp2_ref[0] = xeven
    xp2_ref[1] = xodd
    w = jax.nn.softmax(pw_ref[...], axis=-1)
    xp = xeven * w[:, 0:1] + xodd * w[:, 1:2]
    t = jnp.dot(xp, W_ref[...], preferred_element_type=jnp.float32) + b_ref[...]
    xd = jnp.maximum(_ln(t, g_ref[...], be_ref[...]), 0.0)
    xd_ref[...] = xd
    h = _ln(xd, lng_ref[...], lnb_ref[...])
    h_ref[...] = _pack_bf16(h)
    logits = jnp.dot(h, Wr_ref[...], preferred_element_type=jnp.float32) + br_ref[...]
    dest, be, g0, g1 = _route_compute(logits, NL, PB_LO)
    dest_ref[...] = dest
    bemap_ref[...] = be
    g0_ref[...] = g0
    g1_ref[...] = g1


def _route_body(nt, pb, lg_ref, dest_ref, be_ref, g0_ref, g1_ref):
    dest, be, g0, g1 = _route_compute(lg_ref[...], nt, pb)
    dest_ref[...] = dest
    be_ref[...] = be
    g0_ref[...] = g0
    g1_ref[...] = g1


def _ffn_body(be_ref, x_ref, W1_ref, b1_ref, W2_ref, b2_ref, out_ref):
    b = pl.program_id(0)

    @pl.when(be_ref[b] >= 0)
    def _():
        xin = _unpack_bf16(x_ref[...])
        hidden = jnp.maximum(
            jnp.dot(xin, W1_ref[0], preferred_element_type=jnp.float32)
            + b1_ref[0], 0.0)
        out_ref[...] = _pack_bf16(
            jnp.dot(hidden, W2_ref[0], preferred_element_type=jnp.float32)
            + b2_ref[0])


def _up_body(xd_ref, y0_ref, y1_ref, g0_ref, g1_ref, W1_ref, b1_ref, g_ref,
             be_ref, W2_ref, b2_ref, pos_ref, xp2_ref, lng_ref, lnb_ref,
             Wr_ref, br_ref, s_ref, xc2_ref, h2_ref, lg_ref):
    y0 = _unpack_bf16(y0_ref[...])
    y1 = _unpack_bf16(y1_ref[...])
    xl = xd_ref[...] + g0_ref[...] * y0 + g1_ref[...] * y1
    t1 = jnp.dot(xl, W1_ref[...], preferred_element_type=jnp.float32) + b1_ref[...]
    t1 = jnp.maximum(_ln(t1, g_ref[...], be_ref[...]), 0.0)
    t2 = jnp.dot(t1, W2_ref[...], preferred_element_type=jnp.float32) + b2_ref[...]
    s = s_ref[0, 0]
    xce = xp2_ref[0] + s * (t2[:, :D] + pos_ref[0:1, :])
    xco = xp2_ref[1] + s * (t2[:, D:] + pos_ref[1:2, :])
    xc2_ref[0] = xce
    xc2_ref[1] = xco
    he = _ln(xce, lng_ref[...], lnb_ref[...])
    ho = _ln(xco, lng_ref[...], lnb_ref[...])
    h2_ref[0] = _pack_bf16(he)
    h2_ref[1] = _pack_bf16(ho)
    Wr = Wr_ref[...]
    br = br_ref[...]
    lg_ref[0] = jnp.dot(he, Wr, preferred_element_type=jnp.float32) + br
    lg_ref[1] = jnp.dot(ho, Wr, preferred_element_type=jnp.float32) + br


def _final_body(xp2_ref, xc2_ref, g0e_ref, g0o_ref, g1e_ref, g1o_ref,
                y0e_ref, y0o_ref, y1e_ref, y1o_ref, s_ref, out_ref):
    s = s_ref[0, 0]
    oute = xp2_ref[0] + s * (xc2_ref[0]
                             + g0e_ref[...] * _unpack_bf16(y0e_ref[...])
                             + g1e_ref[...] * _unpack_bf16(y1e_ref[...]))
    outo = xp2_ref[1] + s * (xc2_ref[1]
                             + g0o_ref[...] * _unpack_bf16(y0o_ref[...])
                             + g1o_ref[...] * _unpack_bf16(y1o_ref[...]))
    bf = oute.shape[0]
    out_ref[...] = jnp.stack([oute, outo], axis=1).reshape(2 * bf, D)


def _sc_dispatch(h, dest, pr):
    nt, wd = h.shape
    info = plsc.get_sparse_core_info()
    nc = info.num_cores
    nw = nc * info.num_subcores
    ch = nt // nw
    sub = min(ch, 64)
    mesh = plsc.VectorSubcoreMesh(core_axis_name="c", subcore_axis_name="s")

    @functools.partial(
        pl.kernel, mesh=mesh,
        out_type=jax.ShapeDtypeStruct((pr, wd), h.dtype),
        scratch_types=[
            pltpu.VMEM((sub,), jnp.int32),
            pltpu.VMEM((sub,), jnp.int32),
            pltpu.VMEM((sub, wd), h.dtype),
            pltpu.SemaphoreType.DMA,
            pltpu.SemaphoreType.DMA,
        ],
    )
    def k(h_hbm, dest_hbm, buf_hbm, i0_v, i1_v, rows_v, s0, s1):
        wid = lax.axis_index("s") * nc + lax.axis_index("c")
        base = wid * ch
        for j in range(ch // sub):
            tbase = base + j * sub
            pltpu.sync_copy(dest_hbm.at[pl.ds(tbase, sub)], i0_v)
            pltpu.sync_copy(dest_hbm.at[pl.ds(nt + tbase, sub)], i1_v)
            pltpu.sync_copy(h_hbm.at[pl.ds(tbase, sub)], rows_v)
            c0 = pltpu.async_copy(rows_v, buf_hbm.at[i0_v], s0)
            c1 = pltpu.async_copy(rows_v, buf_hbm.at[i1_v], s1)
            c0.wait()
            c1.wait()

    return k(h, dest)


def _sc_combine(y, dest):
    na = dest.shape[0]
    wd = y.shape[1]
    info = plsc.get_sparse_core_info()
    nc = info.num_cores
    nw = nc * info.num_subcores
    ch = na // nw
    sub = min(ch, 64)
    mesh = plsc.VectorSubcoreMesh(core_axis_name="c", subcore_axis_name="s")

    @functools.partial(
        pl.kernel, mesh=mesh,
        out_type=jax.ShapeDtypeStruct((na, wd), y.dtype),
        scratch_types=[
            pltpu.VMEM((sub,), jnp.int32),
            pltpu.VMEM((sub, wd), y.dtype),
            pltpu.SemaphoreType.DMA,
        ],
    )
    def k(y_hbm, dest_hbm, out_hbm, idx_v, rows_v, sem):
        wid = lax.axis_index("s") * nc + lax.axis_index("c")
        base = wid * ch
        for j in range(ch // sub):
            abase = base + j * sub
            pltpu.sync_copy(dest_hbm.at[pl.ds(abase, sub)], idx_v)
            pltpu.async_copy(y_hbm.at[idx_v], rows_v, sem).wait()
            pltpu.sync_copy(rows_v, out_hbm.at[pl.ds(abase, sub)])

    return k(y, dest)


def _route_call(lg, nt, pb):
    f32, i32 = jnp.float32, jnp.int32
    return pl.pallas_call(
        functools.partial(_route_body, nt, pb),
        out_shape=[jax.ShapeDtypeStruct((2 * nt, 1), i32),
                   jax.ShapeDtypeStruct((pb, 1), i32),
                   jax.ShapeDtypeStruct((nt, 1), f32),
                   jax.ShapeDtypeStruct((nt, 1), f32)],
        interpret=_INTERPRET,
    )(lg)


def _ffn_call(be, buf, W1, b1, W2, b2, pr, pb):
    return pl.pallas_call(
        _ffn_body,
        grid_spec=pltpu.PrefetchScalarGridSpec(
            num_scalar_prefetch=1,
            grid=(pb,),
            in_specs=[
                pl.BlockSpec((BM, D2), lambda b, be: (b, 0)),
                pl.BlockSpec((1, D, DFF),
                             lambda b, be: (jnp.maximum(be[b], 0), 0, 0)),
                pl.BlockSpec((1, 1, DFF),
                             lambda b, be: (jnp.maximum(be[b], 0), 0, 0)),
                pl.BlockSpec((1, DFF, D),
                             lambda b, be: (jnp.maximum(be[b], 0), 0, 0)),
                pl.BlockSpec((1, 1, D),
                             lambda b, be: (jnp.maximum(be[b], 0), 0, 0)),
            ],
            out_specs=pl.BlockSpec((BM, D2), lambda b, be: (b, 0)),
        ),
        out_shape=jax.ShapeDtypeStruct((pr, D2), jnp.int32),
        interpret=_INTERPRET,
    )(be, buf, W1, b1.reshape(E, 1, DFF), W2, b2.reshape(E, 1, D))


def kernel(x, down_pool_w, down_W, down_b, down_g, down_beta, lo_ln_g, lo_ln_b,
           lo_Wr, lo_br, lo_W1, lo_b1, lo_W2, lo_b2, up_W1, up_b1, up_g, up_beta,
           up_W2, up_b2, up_pos, hi_ln_g, hi_ln_b, hi_Wr, hi_br, hi_W1, hi_b1,
           hi_W2, hi_b2, scale_lo, scale_hi):
    f32, i32 = jnp.float32, jnp.int32
    x2 = x.reshape(N, D)
    r2 = lambda v: v.reshape(1, -1)
    sL = jnp.reshape(scale_lo, (1, 1)).astype(f32)
    sH = jnp.reshape(scale_hi, (1, 1)).astype(f32)

    xp2, xd, h_lo, dest_lo, be_lo, g0_lo, g1_lo = pl.pallas_call(
        _down_body,
        out_shape=[jax.ShapeDtypeStruct((2, NL, D), f32),
                   jax.ShapeDtypeStruct((NL, D), f32),
                   jax.ShapeDtypeStruct((NL, D2), i32),
                   jax.ShapeDtypeStruct((2 * NL, 1), i32),
                   jax.ShapeDtypeStruct((PB_LO, 1), i32),
                   jax.ShapeDtypeStruct((NL, 1), f32),
                   jax.ShapeDtypeStruct((NL, 1), f32)],
        interpret=_INTERPRET,
    )(x2, down_pool_w, down_W, r2(down_b), r2(down_g), r2(down_beta),
      r2(lo_ln_g), r2(lo_ln_b), lo_Wr, r2(lo_br))

    buf_lo = _sc_dispatch(h_lo, dest_lo.reshape(-1), PR_LO)
    y_lo = _ffn_call(be_lo.reshape(-1), buf_lo, lo_W1, lo_b1, lo_W2, lo_b2,
                     PR_LO, PB_LO)
    y2_lo = _sc_combine(y_lo, dest_lo.reshape(-1))

    BU = 256
    TU = NL // BU
    blk = lambda: pl.BlockSpec((BU, D), lambda t: (t, 0))
    blk1 = lambda: pl.BlockSpec((BU, 1), lambda t: (t, 0))
    pln = lambda: pl.BlockSpec((2, BU, D), lambda t: (0, t, 0))
    xc2, h2, lg_hi = pl.pallas_call(
        _up_body,
        grid=(TU,),
        in_specs=[
            blk(),
            pl.BlockSpec((BU, D2), lambda t: (t, 0)),
            pl.BlockSpec((BU, D2), lambda t: (NL // BU + t, 0)),
            blk1(), blk1(),
            _full((D, 2 * D)), _full((1, 2 * D)), _full((1, 2 * D)),
            _full((1, 2 * D)), _full((2 * D, 2 * D)), _full((1, 2 * D)),
            _full((2, D)),
            pln(),
            _full((1, D)), _full((1, D)),
            _full((D, E)), _full((1, E)), _full((1, 1)),
        ],
        out_specs=[pln(),
                   pl.BlockSpec((2, BU, D2), lambda t: (0, t, 0)),
                   pl.BlockSpec((2, BU, E), lambda t: (0, t, 0))],
        out_shape=[jax.ShapeDtypeStruct((2, NL, D), f32),
                   jax.ShapeDtypeStruct((2, NL, D2), i32),
                   jax.ShapeDtypeStruct((2, NL, E), f32)],
        interpret=_INTERPRET,
    )(xd, y2_lo, y2_lo, g0_lo, g1_lo, up_W1, r2(up_b1), r2(up_g), r2(up_beta),
      up_W2, r2(up_b2), up_pos, xp2, r2(hi_ln_g), r2(hi_ln_b),
      hi_Wr, r2(hi_br), sL)

    hp = h2.reshape(N, D2)
    dest_hi, be_hi, g0_hi, g1_hi = _route_call(lg_hi.reshape(N, E), N, PB_HI)
    buf_hi = _sc_dispatch(hp, dest_hi.reshape(-1), PR_HI)
    y_hi = _ffn_call(be_hi.reshape(-1), buf_hi, hi_W1, hi_b1, hi_W2, hi_b2,
                     PR_HI, PB_HI)
    y2_hi = _sc_combine(y_hi, dest_hi.reshape(-1))

    BF = 256
    TF = NL // BF
    fy = lambda off: pl.BlockSpec((BF, D2), lambda t: (off + t, 0))
    fb1 = lambda off: pl.BlockSpec((BF, 1), lambda t: (off + t, 0))
    fpl = lambda: pl.BlockSpec((2, BF, D), lambda t: (0, t, 0))
    outp = pl.pallas_call(
        _final_body,
        grid=(TF,),
        in_specs=[
            fpl(), fpl(),
            fb1(0), fb1(NL // BF), fb1(0), fb1(NL // BF),
            fy(0), fy(NL // BF), fy(2 * (NL // BF)), fy(3 * (NL // BF)),
            _full((1, 1)),
        ],
        out_specs=pl.BlockSpec((2 * BF, D), lambda t: (t, 0)),
        out_shape=jax.ShapeDtypeStruct((N, D), f32),
        interpret=_INTERPRET,
    )(xp2, xc2, g0_hi, g0_hi, g1_hi, g1_hi, y2_hi, y2_hi, y2_hi, y2_hi, sH)

    return outp.reshape(1, N, D)


def _full(shape):
    nd = len(shape)
    return pl.BlockSpec(shape, lambda *_: (0,) * nd)
